# Initial kernel scaffold; baseline (speedup 1.0000x reference)
import jax
import jax.numpy as jnp
from jax.experimental import pallas as pl


def kernel(x, edge_index, edge_attr, batch, W_gat, att_src, att_dst, att_edge, W_edge, b_gat, enc_W1, enc_b1, prelu_a, enc_W2, enc_b2, p_W1, p_b1, p_W2, p_b2, p_W3, p_b3):
    raise NotImplementedError("write your pallas kernel here")


# speedup vs baseline: 1.0001x; 1.0001x over previous
import jax
import jax.numpy as jnp
from jax.experimental import pallas as pl

N = 10000
HEADS = 4
C = 32
HID = 128
LATENT = 32
NUM_GRAPHS = 64


def _mlp_body(pooled_ref, w1, b1, w2, b2, w3, b3, out_ref):
    p = pooled_ref[...]
    p = jax.nn.silu(p @ w1[...] + b1[...])
    p = jax.nn.silu(p @ w2[...] + b2[...])
    out_ref[...] = p @ w3[...] + b3[...]


def kernel(x, edge_index, edge_attr, batch, W_gat, att_src, att_dst, att_edge, W_edge, b_gat, enc_W1, enc_b1, prelu_a, enc_W2, enc_b2, p_W1, p_b1, p_W2, p_b2, p_W3, p_b3):
    n = x.shape[0]
    src = edge_index[0]
    dst = edge_index[1]
    deg = jax.ops.segment_sum(jnp.ones((src.shape[0],), jnp.float32), dst, num_segments=n)
    loop_attr = jax.ops.segment_sum(edge_attr, dst, num_segments=n) / jnp.maximum(deg, 1.0)[:, None]
    loop_idx = jnp.arange(n, dtype=src.dtype)
    src = jnp.concatenate([src, loop_idx])
    dst = jnp.concatenate([dst, loop_idx])
    ea = jnp.concatenate([edge_attr, loop_attr], axis=0)
    xp = (x @ W_gat).reshape(n, HEADS, C)
    ep = (ea @ W_edge).reshape(-1, HEADS, C)
    a_src = (xp * att_src[None]).sum(-1)
    a_dst = (xp * att_dst[None]).sum(-1)
    a_edge = (ep * att_edge[None]).sum(-1)
    alpha = a_src[src] + a_dst[dst] + a_edge
    alpha = jax.nn.leaky_relu(alpha, 0.2)
    amax = jax.ops.segment_max(alpha, dst, num_segments=n)
    ex = jnp.exp(alpha - amax[dst])
    denom = jax.ops.segment_sum(ex, dst, num_segments=n)
    att = ex / (denom[dst] + 1e-16)
    msg = xp[src] * att[:, :, None]
    out = jax.ops.segment_sum(msg, dst, num_segments=n).reshape(n, HEADS * C)
    h = jax.nn.elu(out + b_gat)
    h1 = h @ enc_W1 + enc_b1
    h1 = jnp.where(h1 >= 0, h1, prelu_a * h1)
    z = h1 @ enc_W2 + enc_b2
    g = NUM_GRAPHS
    counts = jax.ops.segment_sum(jnp.ones((z.shape[0],), jnp.float32), batch, num_segments=g)
    pooled = jax.ops.segment_sum(z, batch, num_segments=g) / jnp.maximum(counts, 1.0)[:, None]
    pred = pl.pallas_call(
        _mlp_body,
        out_shape=jax.ShapeDtypeStruct((g, 1), jnp.float32),
    )(pooled, p_W1, p_b1[None, :], p_W2, p_b2[None, :], p_W3, p_b3[None, :]).squeeze(-1)
    return (pred, z)


# speedup vs baseline: 18.4984x; 18.4957x over previous
import functools

import jax
import jax.numpy as jnp
from jax import lax
from jax.experimental import pallas as pl
from jax.experimental.pallas import tpu as pltpu
from jax.experimental.pallas import tpu_sc as plsc

N = 10000
E = 320000
NODE_DIM = 128
EDGE_DIM = 16
HEADS = 4
HID = 128
C = HID // HEADS
LATENT = 32
G = 64

NPAD = 10240
EP = 327680
NC = 2
NS = 16
NW = NC * NS
EPT = EP // NW
KA = 64
NCHUNK = EPT // KA
BN = 1024
NBLK = NPAD // BN
RPT = NPAD // NS


def _pre_body(x_ref, w_ref, am_ref, xp_ref, asd_ref):
    xp = x_ref[...] @ w_ref[...]
    xp_ref[...] = xp
    asd_ref[...] = xp @ am_ref[...]


def _aed_body(ea_ref, we_ref, aef_ref, aed_ref):
    ve = we_ref[...] @ aef_ref[...]
    aed_ref[...] = ea_ref[...] @ ve


def _comb_body(na_ref, asd_ref, exl_ref, dinv_ref):
    na = na_ref[0] + na_ref[1]
    deg = jnp.maximum(na[:, 8:9], 1.0)
    asd = asd_ref[...]
    al = asd[:, 0:4] + asd[:, 4:8] + na[:, 4:8] / deg
    al = jnp.where(al >= 0, al, 0.2 * al)
    exl = jnp.exp(al)
    exl_ref[...] = exl
    dinv_ref[...] = 1.0 / (na[:, 0:4] + exl + 1e-16)


def _silu(v):
    return v / (1.0 + jnp.exp(-v))


def _tail_body(p_ref, xp_ref, exl_ref, dinv_ref, em_ref, bg_ref, b3_ref,
               w1_ref, b1_ref, pa_ref, w2_ref, b2_ref, pw1_ref, pb1_ref,
               pw2_ref, pb2_ref, pw3_ref, pb3_ref, z_ref, pred_ref, ps_ref,
               cnt_ref):
    i = pl.program_id(0)
    exl = exl_ref[...] @ em_ref[...]
    dinv = dinv_ref[...] @ em_ref[...]
    out = (p_ref[0] + p_ref[1] + xp_ref[...] * exl) * dinv + bg_ref[...]
    h = jnp.where(out > 0, out, jnp.exp(jnp.minimum(out, 0.0)) - 1.0)
    h1 = h @ w1_ref[...] + b1_ref[...]
    h1 = jnp.where(h1 >= 0, h1, pa_ref[...] * h1)
    z = h1 @ w2_ref[...] + b2_ref[...]
    z_ref[...] = z
    bids = b3_ref[0]
    gi = lax.broadcasted_iota(jnp.int32, (G, 1), 0)
    mask = (bids == gi).astype(jnp.float32)
    ps = mask @ z
    ct = jnp.sum(mask, axis=1, keepdims=True)

    @pl.when(i == 0)
    def _():
        ps_ref[...] = ps
        cnt_ref[...] = ct

    @pl.when(i > 0)
    def _():
        ps_ref[...] += ps
        cnt_ref[...] += ct

    @pl.when(i == pl.num_programs(0) - 1)
    def _():
        pooled = ps_ref[...] / jnp.maximum(cnt_ref[...], 1.0)
        p1 = _silu(pooled @ pw1_ref[...] + pb1_ref[...])
        p2 = _silu(p1 @ pw2_ref[...] + pb2_ref[...])
        pred_ref[...] = p2 @ pw3_ref[...] + pb3_ref[...]


_MESH = plsc.VectorSubcoreMesh(core_axis_name="c", subcore_axis_name="s",
                               num_cores=NC, num_subcores=NS)
_SC_PARAMS = pltpu.CompilerParams(needs_layout_passes=False)


@functools.partial(
    pl.kernel,
    out_type=[jax.ShapeDtypeStruct((NC, NPAD, HID), jnp.float32),
              jax.ShapeDtypeStruct((NC, NPAD, HID), jnp.float32),
              jax.ShapeDtypeStruct((EP * 4,), jnp.float32)],
    mesh=_MESH,
    scratch_types=[
        pltpu.VMEM((1, KA), jnp.int32),
        pltpu.VMEM((1, KA), jnp.int32),
        pltpu.VMEM((KA * 4,), jnp.float32),
        pltpu.VMEM((KA * 4,), jnp.float32),
        pltpu.VMEM((KA, HID), jnp.float32),
        pltpu.VMEM((KA, HID), jnp.float32),
        pltpu.VMEM((KA, HID), jnp.float32),
        pltpu.VMEM((64,), jnp.float32),
        pltpu.VMEM_SHARED((NPAD, HID), jnp.float32),
        pltpu.SemaphoreType.DMA,
    ],
    compiler_params=_SC_PARAMS,
)
def _sc_edges(src_hbm, dst_hbm, aed_hbm, asd_hbm, xp_hbm, z128_hbm,
              nacc_out, part_out, ex_out,
              src_v, dst_v, aed_v, ex_v, srows_v, drows_v, rows_v, w_v,
              acc_sh, sem):
    cid = lax.axis_index("c")
    sid = lax.axis_index("s")
    wid = cid * NS + sid
    r0 = sid * RPT

    i16 = lax.iota(jnp.int32, 16)
    one16 = jnp.ones((16,), jnp.float32)
    zero16 = jnp.zeros((16,), jnp.float32)

    pltpu.sync_copy(z128_hbm.at[pl.ds(r0, RPT)], acc_sh.at[pl.ds(r0, RPT)])

    def zrow_body(r, carry):
        for c8 in range(8):
            rows_v[r, pl.ds(c8 * 16, 16)] = zero16
        return carry

    lax.fori_loop(0, KA, zrow_body, 0)

    def pinit_body(g, carry):
        l16 = g * 16 + i16
        plsc.store_scatter(rows_v, [l16, jnp.full((16,), 8, jnp.int32)],
                           one16)
        return carry

    lax.fori_loop(0, KA // 16, pinit_body, 0)
    plsc.subcore_barrier()

    def a_group(g, carry):
        l16 = g * 16 + i16
        for h in range(HEADS):
            asrc = plsc.load_gather(srows_v, [l16, jnp.full((16,), h,
                                                            jnp.int32)])
            adst = plsc.load_gather(drows_v, [l16, jnp.full((16,), 4 + h,
                                                            jnp.int32)])
            aeh = plsc.load_gather(aed_v, [l16 * 4 + h])
            al = asrc + adst + aeh
            al = jnp.where(al >= 0, al, 0.2 * al)
            exh = jnp.exp(al)
            plsc.store_scatter(ex_v, [l16 * 4 + h], exh)
            plsc.store_scatter(rows_v, [l16, jnp.full((16,), h, jnp.int32)],
                               exh)
            plsc.store_scatter(rows_v, [l16, jnp.full((16,), 4 + h,
                                                      jnp.int32)], aeh)
        return carry

    def a_chunk(k, carry):
        ebase = wid * EPT + k * KA
        pltpu.sync_copy(src_hbm.at[wid * NCHUNK + k], src_v)
        pltpu.sync_copy(dst_hbm.at[wid * NCHUNK + k], dst_v)
        pltpu.sync_copy(aed_hbm.at[pl.ds(ebase * 4, KA * 4)], aed_v)
        cp1 = pltpu.async_copy(asd_hbm.at[src_v.at[0]], srows_v, sem)
        cp2 = pltpu.async_copy(asd_hbm.at[dst_v.at[0]], drows_v, sem)
        cp1.wait()
        cp2.wait()
        lax.fori_loop(0, KA // 16, a_group, 0)
        pltpu.sync_copy(ex_v, ex_out.at[pl.ds(ebase * 4, KA * 4)])
        pltpu.sync_copy(rows_v, acc_sh.at[dst_v.at[0]], add=True)
        return carry

    lax.fori_loop(0, NCHUNK, a_chunk, 0)
    plsc.subcore_barrier()

    pltpu.sync_copy(acc_sh.at[pl.ds(r0, RPT)],
                    nacc_out.at[cid, pl.ds(r0, RPT)])
    pltpu.sync_copy(z128_hbm.at[pl.ds(r0, RPT)], acc_sh.at[pl.ds(r0, RPT)])
    plsc.subcore_barrier()

    def b_group(g, carry):
        for i in range(16):
            eloc = g * 16 + i
            for h in range(HEADS):
                wb = plsc.load_gather(
                    ex_v, [g * 64 + jnp.full((16,), i * 4 + h, jnp.int32)])
                for half in range(2):
                    jc = 2 * h + half
                    rv = rows_v[eloc, pl.ds(jc * 16, 16)]
                    rows_v[eloc, pl.ds(jc * 16, 16)] = rv * wb
        return carry

    def b_chunk(k, carry):
        ebase = wid * EPT + k * KA
        pltpu.sync_copy(src_hbm.at[wid * NCHUNK + k], src_v)
        pltpu.sync_copy(dst_hbm.at[wid * NCHUNK + k], dst_v)
        pltpu.sync_copy(ex_out.at[pl.ds(ebase * 4, KA * 4)], ex_v)
        pltpu.async_copy(xp_hbm.at[src_v.at[0]], rows_v, sem).wait()
        lax.fori_loop(0, KA // 16, b_group, 0)
        pltpu.sync_copy(rows_v, acc_sh.at[dst_v.at[0]], add=True)
        return carry

    lax.fori_loop(0, NCHUNK, b_chunk, 0)
    plsc.subcore_barrier()
    pltpu.sync_copy(acc_sh.at[pl.ds(r0, RPT)],
                    part_out.at[cid, pl.ds(r0, RPT)])


def kernel(x, edge_index, edge_attr, batch, W_gat, att_src, att_dst, att_edge,
           W_edge, b_gat, enc_W1, enc_b1, prelu_a, enc_W2, enc_b2, p_W1, p_b1,
           p_W2, p_b2, p_W3, p_b3):
    f32 = jnp.float32
    x_pad = jnp.pad(x, ((0, NPAD - N), (0, 0)))
    src3d = jnp.pad(edge_index[0], (0, EP - E)).reshape(EP // KA, 1, KA)
    dst3d = jnp.pad(edge_index[1], (0, EP - E),
                    constant_values=N).reshape(EP // KA, 1, KA)
    ea_pad = jnp.pad(edge_attr, ((0, EP - E), (0, 0)))
    batch3 = jnp.pad(batch, (0, NPAD - N),
                     constant_values=G).reshape(NBLK, 1, BN)

    rep = jnp.repeat(jnp.eye(HEADS, dtype=f32), C, axis=0)
    aef = rep * att_edge.reshape(-1)[:, None]
    att_mat = jnp.concatenate([rep * att_src.reshape(-1)[:, None],
                               rep * att_dst.reshape(-1)[:, None]], axis=1)
    att_mat128 = jnp.pad(att_mat, ((0, 0), (0, 120)))
    zeros128 = jnp.zeros((NPAD, HID), f32)

    xp, asd16 = pl.pallas_call(
        _pre_body,
        grid=(NBLK,),
        in_specs=[pl.BlockSpec((BN, NODE_DIM), lambda i: (i, 0)),
                  pl.BlockSpec((NODE_DIM, HID), lambda i: (0, 0)),
                  pl.BlockSpec((HID, HID), lambda i: (0, 0))],
        out_specs=[pl.BlockSpec((BN, HID), lambda i: (i, 0)),
                   pl.BlockSpec((BN, HID), lambda i: (i, 0))],
        out_shape=[jax.ShapeDtypeStruct((NPAD, HID), f32),
                   jax.ShapeDtypeStruct((NPAD, HID), f32)],
    )(x_pad, W_gat, att_mat128)

    EB = 2048
    aed = pl.pallas_call(
        _aed_body,
        grid=(EP // EB,),
        in_specs=[pl.BlockSpec((EB, EDGE_DIM), lambda i: (i, 0)),
                  pl.BlockSpec((EDGE_DIM, HID), lambda i: (0, 0)),
                  pl.BlockSpec((HID, HEADS), lambda i: (0, 0))],
        out_specs=pl.BlockSpec((EB, HEADS), lambda i: (i, 0)),
        out_shape=jax.ShapeDtypeStruct((EP, HEADS), f32),
    )(ea_pad, W_edge, aef)

    nacc, part, _ = _sc_edges(src3d, dst3d, aed.reshape(EP * 4), asd16, xp,
                              zeros128)

    exl, dinv = pl.pallas_call(
        _comb_body,
        grid=(NBLK,),
        in_specs=[pl.BlockSpec((NC, BN, HID), lambda i: (0, i, 0)),
                  pl.BlockSpec((BN, HID), lambda i: (i, 0))],
        out_specs=[pl.BlockSpec((BN, HEADS), lambda i: (i, 0)),
                   pl.BlockSpec((BN, HEADS), lambda i: (i, 0))],
        out_shape=[jax.ShapeDtypeStruct((NPAD, HEADS), f32),
                   jax.ShapeDtypeStruct((NPAD, HEADS), f32)],
    )(nacc, asd16)

    em = jnp.repeat(jnp.eye(HEADS, dtype=f32), C, axis=1)
    z_pad, pred = pl.pallas_call(
        _tail_body,
        grid=(NBLK,),
        in_specs=[pl.BlockSpec((NC, BN, HID), lambda i: (0, i, 0)),
                  pl.BlockSpec((BN, HID), lambda i: (i, 0)),
                  pl.BlockSpec((BN, HEADS), lambda i: (i, 0)),
                  pl.BlockSpec((BN, HEADS), lambda i: (i, 0)),
                  pl.BlockSpec((HEADS, HID), lambda i: (0, 0)),
                  pl.BlockSpec((1, HID), lambda i: (0, 0)),
                  pl.BlockSpec((1, 1, BN), lambda i: (i, 0, 0)),
                  pl.BlockSpec((HID, HID), lambda i: (0, 0)),
                  pl.BlockSpec((1, HID), lambda i: (0, 0)),
                  pl.BlockSpec((1, 1), lambda i: (0, 0)),
                  pl.BlockSpec((HID, LATENT), lambda i: (0, 0)),
                  pl.BlockSpec((1, LATENT), lambda i: (0, 0)),
                  pl.BlockSpec((LATENT, 128), lambda i: (0, 0)),
                  pl.BlockSpec((1, 128), lambda i: (0, 0)),
                  pl.BlockSpec((128, 64), lambda i: (0, 0)),
                  pl.BlockSpec((1, 64), lambda i: (0, 0)),
                  pl.BlockSpec((64, 1), lambda i: (0, 0)),
                  pl.BlockSpec((1, 1), lambda i: (0, 0))],
        out_specs=[pl.BlockSpec((BN, LATENT), lambda i: (i, 0)),
                   pl.BlockSpec((G, 1), lambda i: (0, 0))],
        out_shape=[jax.ShapeDtypeStruct((NPAD, LATENT), f32),
                   jax.ShapeDtypeStruct((G, 1), f32)],
        scratch_shapes=[pltpu.VMEM((G, LATENT), f32),
                        pltpu.VMEM((G, 1), f32)],
    )(part, xp, exl, dinv, em, b_gat.reshape(1, HID), batch3, enc_W1,
      enc_b1.reshape(1, HID), prelu_a.reshape(1, 1), enc_W2,
      enc_b2.reshape(1, LATENT), p_W1, p_b1.reshape(1, 128), p_W2,
      p_b2.reshape(1, 64), p_W3, p_b3.reshape(1, 1))

    return (pred.reshape(G), z_pad[:N])


# speedup vs baseline: 20.5311x; 1.1099x over previous
import functools

import jax
import jax.numpy as jnp
from jax import lax
from jax.experimental import pallas as pl
from jax.experimental.pallas import tpu as pltpu
from jax.experimental.pallas import tpu_sc as plsc

N = 10000
E = 320000
NODE_DIM = 128
EDGE_DIM = 16
HEADS = 4
HID = 128
C = HID // HEADS
LATENT = 32
G = 64

NPAD = 10240
EP = 327680
NC = 2
NS = 16
NW = NC * NS
EPT = EP // NW
KA = 64
NCHUNK = EPT // KA
KB = 128
NCHUNK_B = EPT // KB
BN = 1024
NBLK = NPAD // BN
RPT = NPAD // NS


def _pre_body(x_ref, w_ref, am_ref, xp_ref, asd_ref):
    xp = x_ref[...] @ w_ref[...]
    xp_ref[...] = xp
    asd_ref[...] = xp @ am_ref[...]


def _aed_body(ea_ref, we_ref, aef_ref, aed_ref):
    ve = we_ref[...] @ aef_ref[...]
    aed_ref[...] = ea_ref[...] @ ve


def _comb_body(na_ref, asd_ref, exl_ref, dinv_ref):
    na = na_ref[0] + na_ref[1]
    deg = jnp.maximum(na[:, 8:9], 1.0)
    asd = asd_ref[...]
    al = asd[:, 0:4] + asd[:, 4:8] + na[:, 4:8] / deg
    al = jnp.where(al >= 0, al, 0.2 * al)
    exl = jnp.exp(al)
    exl_ref[...] = exl
    dinv_ref[...] = 1.0 / (na[:, 0:4] + exl + 1e-16)


def _silu(v):
    return v / (1.0 + jnp.exp(-v))


def _tail_body(p_ref, xp_ref, exl_ref, dinv_ref, em_ref, bg_ref, b3_ref,
               w1_ref, b1_ref, pa_ref, w2_ref, b2_ref, pw1_ref, pb1_ref,
               pw2_ref, pb2_ref, pw3_ref, pb3_ref, z_ref, pred_ref, ps_ref,
               cnt_ref):
    i = pl.program_id(0)
    exl = exl_ref[...] @ em_ref[...]
    dinv = dinv_ref[...] @ em_ref[...]
    out = (p_ref[0] + p_ref[1] + xp_ref[...] * exl) * dinv + bg_ref[...]
    h = jnp.where(out > 0, out, jnp.exp(jnp.minimum(out, 0.0)) - 1.0)
    h1 = h @ w1_ref[...] + b1_ref[...]
    h1 = jnp.where(h1 >= 0, h1, pa_ref[...] * h1)
    z = h1 @ w2_ref[...] + b2_ref[...]
    z_ref[...] = z
    bids = b3_ref[0]
    gi = lax.broadcasted_iota(jnp.int32, (G, 1), 0)
    mask = (bids == gi).astype(jnp.float32)
    ps = mask @ z
    ct = jnp.sum(mask, axis=1, keepdims=True)

    @pl.when(i == 0)
    def _():
        ps_ref[...] = ps
        cnt_ref[...] = ct

    @pl.when(i > 0)
    def _():
        ps_ref[...] += ps
        cnt_ref[...] += ct

    @pl.when(i == pl.num_programs(0) - 1)
    def _():
        pooled = ps_ref[...] / jnp.maximum(cnt_ref[...], 1.0)
        p1 = _silu(pooled @ pw1_ref[...] + pb1_ref[...])
        p2 = _silu(p1 @ pw2_ref[...] + pb2_ref[...])
        pred_ref[...] = p2 @ pw3_ref[...] + pb3_ref[...]


_MESH = plsc.VectorSubcoreMesh(core_axis_name="c", subcore_axis_name="s",
                               num_cores=NC, num_subcores=NS)
_SC_PARAMS = pltpu.CompilerParams(needs_layout_passes=False)


@functools.partial(
    pl.kernel,
    out_type=[jax.ShapeDtypeStruct((NC, NPAD, HID), jnp.float32),
              jax.ShapeDtypeStruct((NC, NPAD, HID), jnp.float32),
              jax.ShapeDtypeStruct((EP * 4,), jnp.float32)],
    mesh=_MESH,
    scratch_types=[
        pltpu.VMEM((2, KA), jnp.int32),
        pltpu.VMEM((2, KB), jnp.int32),
        pltpu.VMEM((KA * 4,), jnp.float32),
        pltpu.VMEM((KB * 4,), jnp.float32),
        pltpu.VMEM((KA, HID), jnp.float32),
        pltpu.VMEM((KA, HID), jnp.float32),
        pltpu.VMEM((KB, HID), jnp.float32),
        pltpu.VMEM_SHARED((NPAD, HID), jnp.float32),
        pltpu.SemaphoreType.DMA,
    ],
    compiler_params=_SC_PARAMS,
)
def _sc_edges(sda_hbm, sdb_hbm, aed_hbm, asd_hbm, xp_hbm, z128_hbm,
              nacc_out, part_out, ex_out,
              sda_v, sdb_v, aed_v, ex_v, srows_v, drows_v, rows_v,
              acc_sh, sem):
    cid = lax.axis_index("c")
    sid = lax.axis_index("s")
    wid = cid * NS + sid
    r0 = sid * RPT

    i16 = lax.iota(jnp.int32, 16)
    one16 = jnp.ones((16,), jnp.float32)
    zero16 = jnp.zeros((16,), jnp.float32)

    pltpu.sync_copy(z128_hbm.at[pl.ds(r0, RPT)], acc_sh.at[pl.ds(r0, RPT)])

    def zrow_body(r, carry):
        for c8 in range(8):
            rows_v[r, pl.ds(c8 * 16, 16)] = zero16
        return carry

    lax.fori_loop(0, KB, zrow_body, 0)

    def pinit_body(g, carry):
        l16 = g * 16 + i16
        plsc.store_scatter(rows_v, [l16, jnp.full((16,), 8, jnp.int32)],
                           one16)
        return carry

    lax.fori_loop(0, KA // 16, pinit_body, 0)
    plsc.subcore_barrier()

    def a_group(g, carry):
        l16 = g * 16 + i16
        for h in range(HEADS):
            asrc = plsc.load_gather(srows_v, [l16, jnp.full((16,), h,
                                                            jnp.int32)])
            adst = plsc.load_gather(drows_v, [l16, jnp.full((16,), 4 + h,
                                                            jnp.int32)])
            aeh = plsc.load_gather(aed_v, [l16 * 4 + h])
            al = asrc + adst + aeh
            al = jnp.where(al >= 0, al, 0.2 * al)
            exh = jnp.exp(al)
            plsc.store_scatter(ex_v, [l16 * 4 + h], exh)
            plsc.store_scatter(rows_v, [l16, jnp.full((16,), h, jnp.int32)],
                               exh)
            plsc.store_scatter(rows_v, [l16, jnp.full((16,), 4 + h,
                                                      jnp.int32)], aeh)
        return carry

    def a_chunk(k, carry):
        ebase = wid * EPT + k * KA
        pltpu.sync_copy(sda_hbm.at[wid * NCHUNK + k], sda_v)
        cp0 = pltpu.async_copy(aed_hbm.at[pl.ds(ebase * 4, KA * 4)], aed_v,
                               sem)
        cp1 = pltpu.async_copy(asd_hbm.at[sda_v.at[0]], srows_v, sem)
        cp2 = pltpu.async_copy(asd_hbm.at[sda_v.at[1]], drows_v, sem)
        cp0.wait()
        cp1.wait()
        cp2.wait()
        lax.fori_loop(0, KA // 16, a_group, 0)
        pltpu.sync_copy(ex_v.at[pl.ds(0, KA * 4)],
                        ex_out.at[pl.ds(ebase * 4, KA * 4)])
        pltpu.sync_copy(rows_v.at[pl.ds(0, KA)], acc_sh.at[sda_v.at[1]],
                        add=True)
        return carry

    lax.fori_loop(0, NCHUNK, a_chunk, 0)
    plsc.subcore_barrier()

    pltpu.sync_copy(acc_sh.at[pl.ds(r0, RPT)],
                    nacc_out.at[cid, pl.ds(r0, RPT)])
    pltpu.sync_copy(z128_hbm.at[pl.ds(r0, RPT)], acc_sh.at[pl.ds(r0, RPT)])
    plsc.subcore_barrier()

    def b_group(g, carry):
        for i in range(16):
            eloc = g * 16 + i
            for h in range(HEADS):
                wb = plsc.load_gather(
                    ex_v, [g * 64 + jnp.full((16,), i * 4 + h, jnp.int32)])
                for half in range(2):
                    jc = 2 * h + half
                    rv = rows_v[eloc, pl.ds(jc * 16, 16)]
                    rows_v[eloc, pl.ds(jc * 16, 16)] = rv * wb
        return carry

    def b_chunk(k, carry):
        ebase = wid * EPT + k * KB
        pltpu.sync_copy(sdb_hbm.at[wid * NCHUNK_B + k], sdb_v)
        cp0 = pltpu.async_copy(ex_out.at[pl.ds(ebase * 4, KB * 4)], ex_v,
                               sem)
        cp1 = pltpu.async_copy(xp_hbm.at[sdb_v.at[0]], rows_v, sem)
        cp0.wait()
        cp1.wait()
        lax.fori_loop(0, KB // 16, b_group, 0)
        pltpu.sync_copy(rows_v, acc_sh.at[sdb_v.at[1]], add=True)
        return carry

    lax.fori_loop(0, NCHUNK_B, b_chunk, 0)
    plsc.subcore_barrier()
    pltpu.sync_copy(acc_sh.at[pl.ds(r0, RPT)],
                    part_out.at[cid, pl.ds(r0, RPT)])


def kernel(x, edge_index, edge_attr, batch, W_gat, att_src, att_dst, att_edge,
           W_edge, b_gat, enc_W1, enc_b1, prelu_a, enc_W2, enc_b2, p_W1, p_b1,
           p_W2, p_b2, p_W3, p_b3):
    f32 = jnp.float32
    x_pad = jnp.pad(x, ((0, NPAD - N), (0, 0)))
    src_pad = jnp.pad(edge_index[0], (0, EP - E))
    dummy = N + jnp.arange(EP - E, dtype=jnp.int32) % (NPAD - N)
    dst_pad = jnp.concatenate([edge_index[1], dummy])
    sda = jnp.stack([src_pad.reshape(EP // KA, KA),
                     dst_pad.reshape(EP // KA, KA)], axis=1)
    sdb = jnp.stack([src_pad.reshape(EP // KB, KB),
                     dst_pad.reshape(EP // KB, KB)], axis=1)
    ea_pad = jnp.pad(edge_attr, ((0, EP - E), (0, 0)))
    batch3 = jnp.pad(batch, (0, NPAD - N),
                     constant_values=G).reshape(NBLK, 1, BN)

    rep = jnp.repeat(jnp.eye(HEADS, dtype=f32), C, axis=0)
    aef = rep * att_edge.reshape(-1)[:, None]
    att_mat = jnp.concatenate([rep * att_src.reshape(-1)[:, None],
                               rep * att_dst.reshape(-1)[:, None]], axis=1)
    att_mat128 = jnp.pad(att_mat, ((0, 0), (0, 120)))
    zeros128 = jnp.zeros((NPAD, HID), f32)

    xp, asd16 = pl.pallas_call(
        _pre_body,
        grid=(NBLK,),
        in_specs=[pl.BlockSpec((BN, NODE_DIM), lambda i: (i, 0)),
                  pl.BlockSpec((NODE_DIM, HID), lambda i: (0, 0)),
                  pl.BlockSpec((HID, HID), lambda i: (0, 0))],
        out_specs=[pl.BlockSpec((BN, HID), lambda i: (i, 0)),
                   pl.BlockSpec((BN, HID), lambda i: (i, 0))],
        out_shape=[jax.ShapeDtypeStruct((NPAD, HID), f32),
                   jax.ShapeDtypeStruct((NPAD, HID), f32)],
    )(x_pad, W_gat, att_mat128)

    EB = 2048
    aed = pl.pallas_call(
        _aed_body,
        grid=(EP // EB,),
        in_specs=[pl.BlockSpec((EB, EDGE_DIM), lambda i: (i, 0)),
                  pl.BlockSpec((EDGE_DIM, HID), lambda i: (0, 0)),
                  pl.BlockSpec((HID, HEADS), lambda i: (0, 0))],
        out_specs=pl.BlockSpec((EB, HEADS), lambda i: (i, 0)),
        out_shape=jax.ShapeDtypeStruct((EP, HEADS), f32),
    )(ea_pad, W_edge, aef)

    nacc, part, _ = _sc_edges(sda, sdb, aed.reshape(EP * 4), asd16, xp,
                              zeros128)

    exl, dinv = pl.pallas_call(
        _comb_body,
        grid=(NBLK,),
        in_specs=[pl.BlockSpec((NC, BN, HID), lambda i: (0, i, 0)),
                  pl.BlockSpec((BN, HID), lambda i: (i, 0))],
        out_specs=[pl.BlockSpec((BN, HEADS), lambda i: (i, 0)),
                   pl.BlockSpec((BN, HEADS), lambda i: (i, 0))],
        out_shape=[jax.ShapeDtypeStruct((NPAD, HEADS), f32),
                   jax.ShapeDtypeStruct((NPAD, HEADS), f32)],
    )(nacc, asd16)

    em = jnp.repeat(jnp.eye(HEADS, dtype=f32), C, axis=1)
    z_pad, pred = pl.pallas_call(
        _tail_body,
        grid=(NBLK,),
        in_specs=[pl.BlockSpec((NC, BN, HID), lambda i: (0, i, 0)),
                  pl.BlockSpec((BN, HID), lambda i: (i, 0)),
                  pl.BlockSpec((BN, HEADS), lambda i: (i, 0)),
                  pl.BlockSpec((BN, HEADS), lambda i: (i, 0)),
                  pl.BlockSpec((HEADS, HID), lambda i: (0, 0)),
                  pl.BlockSpec((1, HID), lambda i: (0, 0)),
                  pl.BlockSpec((1, 1, BN), lambda i: (i, 0, 0)),
                  pl.BlockSpec((HID, HID), lambda i: (0, 0)),
                  pl.BlockSpec((1, HID), lambda i: (0, 0)),
                  pl.BlockSpec((1, 1), lambda i: (0, 0)),
                  pl.BlockSpec((HID, LATENT), lambda i: (0, 0)),
                  pl.BlockSpec((1, LATENT), lambda i: (0, 0)),
                  pl.BlockSpec((LATENT, 128), lambda i: (0, 0)),
                  pl.BlockSpec((1, 128), lambda i: (0, 0)),
                  pl.BlockSpec((128, 64), lambda i: (0, 0)),
                  pl.BlockSpec((1, 64), lambda i: (0, 0)),
                  pl.BlockSpec((64, 1), lambda i: (0, 0)),
                  pl.BlockSpec((1, 1), lambda i: (0, 0))],
        out_specs=[pl.BlockSpec((BN, LATENT), lambda i: (i, 0)),
                   pl.BlockSpec((G, 1), lambda i: (0, 0))],
        out_shape=[jax.ShapeDtypeStruct((NPAD, LATENT), f32),
                   jax.ShapeDtypeStruct((G, 1), f32)],
        scratch_shapes=[pltpu.VMEM((G, LATENT), f32),
                        pltpu.VMEM((G, 1), f32)],
    )(part, xp, exl, dinv, em, b_gat.reshape(1, HID), batch3, enc_W1,
      enc_b1.reshape(1, HID), prelu_a.reshape(1, 1), enc_W2,
      enc_b2.reshape(1, LATENT), p_W1, p_b1.reshape(1, 128), p_W2,
      p_b2.reshape(1, 64), p_W3, p_b3.reshape(1, 1))

    return (pred.reshape(G), z_pad[:N])


# speedup vs baseline: 21.6056x; 1.0523x over previous
import functools

import jax
import jax.numpy as jnp
from jax import lax
from jax.experimental import pallas as pl
from jax.experimental.pallas import tpu as pltpu
from jax.experimental.pallas import tpu_sc as plsc

N = 10000
E = 320000
NODE_DIM = 128
EDGE_DIM = 16
HEADS = 4
HID = 128
C = HID // HEADS
LATENT = 32
G = 64

NPAD = 10240
EP = 327680
NC = 2
NS = 16
NW = NC * NS
EPT = EP // NW
KA = 64
NCHUNK = EPT // KA
KB = 128
NCHUNK_B = EPT // KB
BN = 1024
NBLK = NPAD // BN
RPT = NPAD // NS


def _pre_body(x_ref, w_ref, am_ref, xp_ref, asd_ref):
    xp = x_ref[...] @ w_ref[...]
    xp_ref[...] = xp
    asd_ref[...] = xp @ am_ref[...]


def _aed_body(ea_ref, we_ref, aef_ref, aed_ref):
    ve = we_ref[...] @ aef_ref[...]
    aed_ref[...] = ea_ref[...] @ ve


def _comb_body(na_ref, asd_ref, exl_ref, dinv_ref):
    na = na_ref[0] + na_ref[1]
    deg = jnp.maximum(na[:, 12:13], 1.0)
    asd = asd_ref[...]
    al = asd[:, 0:4] + asd[:, 4:8] + na[:, 8:12] / deg
    al = jnp.where(al >= 0, al, 0.2 * al)
    exl = jnp.exp(al)
    exl_ref[...] = exl
    dinv_ref[...] = 1.0 / (na[:, 0:4] + exl + 1e-16)


def _silu(v):
    return v / (1.0 + jnp.exp(-v))


def _tail_body(p_ref, xp_ref, exl_ref, dinv_ref, em_ref, bg_ref, b3_ref,
               w1_ref, b1_ref, pa_ref, w2_ref, b2_ref, pw1_ref, pb1_ref,
               pw2_ref, pb2_ref, pw3_ref, pb3_ref, z_ref, pred_ref, ps_ref,
               cnt_ref):
    i = pl.program_id(0)
    exl = exl_ref[...] @ em_ref[...]
    dinv = dinv_ref[...] @ em_ref[...]
    out = (p_ref[0] + p_ref[1] + xp_ref[...] * exl) * dinv + bg_ref[...]
    h = jnp.where(out > 0, out, jnp.exp(jnp.minimum(out, 0.0)) - 1.0)
    h1 = h @ w1_ref[...] + b1_ref[...]
    h1 = jnp.where(h1 >= 0, h1, pa_ref[...] * h1)
    z = h1 @ w2_ref[...] + b2_ref[...]
    z_ref[...] = z
    bids = b3_ref[0]
    gi = lax.broadcasted_iota(jnp.int32, (G, 1), 0)
    mask = (bids == gi).astype(jnp.float32)
    ps = mask @ z
    ct = jnp.sum(mask, axis=1, keepdims=True)

    @pl.when(i == 0)
    def _():
        ps_ref[...] = ps
        cnt_ref[...] = ct

    @pl.when(i > 0)
    def _():
        ps_ref[...] += ps
        cnt_ref[...] += ct

    @pl.when(i == pl.num_programs(0) - 1)
    def _():
        pooled = ps_ref[...] / jnp.maximum(cnt_ref[...], 1.0)
        p1 = _silu(pooled @ pw1_ref[...] + pb1_ref[...])
        p2 = _silu(p1 @ pw2_ref[...] + pb2_ref[...])
        pred_ref[...] = p2 @ pw3_ref[...] + pb3_ref[...]


_MESH = plsc.VectorSubcoreMesh(core_axis_name="c", subcore_axis_name="s",
                               num_cores=NC, num_subcores=NS)
_SC_PARAMS = pltpu.CompilerParams(needs_layout_passes=False)


@functools.partial(
    pl.kernel,
    out_type=[jax.ShapeDtypeStruct((NC, NPAD, HID), jnp.float32),
              jax.ShapeDtypeStruct((NC, NPAD, HID), jnp.float32),
              jax.ShapeDtypeStruct((EP * 4,), jnp.float32)],
    mesh=_MESH,
    scratch_types=[
        pltpu.VMEM((8, 2, KA), jnp.int32),
        pltpu.VMEM((8, 2, KB), jnp.int32),
        pltpu.VMEM((8 * KA * 4,), jnp.float32),
        pltpu.VMEM((8 * KB * 4,), jnp.float32),
        pltpu.VMEM((KA, HID), jnp.float32),
        pltpu.VMEM((KA, HID), jnp.float32),
        pltpu.VMEM((KB, HID), jnp.float32),
        pltpu.VMEM_SHARED((NPAD, HID), jnp.float32),
        pltpu.SemaphoreType.DMA,
    ],
    compiler_params=_SC_PARAMS,
)
def _sc_edges(sda_hbm, sdb_hbm, aed_hbm, asd_hbm, xp_hbm, z128_hbm,
              nacc_out, part_out, ex_out,
              sda_v, sdb_v, aed_v, ex_v, srows_v, drows_v, rows_v,
              acc_sh, sem):
    cid = lax.axis_index("c")
    sid = lax.axis_index("s")
    wid = cid * NS + sid
    r0 = sid * RPT

    i16 = lax.iota(jnp.int32, 16)
    one16 = jnp.ones((16,), jnp.float32)

    pltpu.sync_copy(z128_hbm.at[pl.ds(r0, RPT)], acc_sh.at[pl.ds(r0, RPT)])
    plsc.subcore_barrier()

    def a_super(s, carry):
        ebase = wid * EPT + s * 8 * KA

        def a_group(jg, carry2):
            j = jg // (KA // 16)
            g = jg % (KA // 16)
            l16 = g * 16 + i16
            asrcs = []
            adsts = []
            aehs = []
            for h in range(HEADS):
                asrcs.append(plsc.load_gather(
                    srows_v, [l16, jnp.full((16,), h, jnp.int32)]))
                adsts.append(plsc.load_gather(
                    drows_v, [l16, jnp.full((16,), 4 + h, jnp.int32)]))
                aehs.append(plsc.load_gather(
                    aed_v, [j * (KA * 4) + l16 * 4 + h]))
            plsc.store_scatter(drows_v, [l16, jnp.full((16,), 12, jnp.int32)],
                               one16)
            for h in range(HEADS):
                al = asrcs[h] + adsts[h] + aehs[h]
                al = jnp.where(al >= 0, al, 0.2 * al)
                exh = jnp.exp(al)
                plsc.store_scatter(ex_v, [j * (KA * 4) + l16 * 4 + h], exh)
                plsc.store_scatter(drows_v, [l16, jnp.full((16,), h,
                                                           jnp.int32)], exh)
                plsc.store_scatter(drows_v, [l16, jnp.full((16,), 8 + h,
                                                           jnp.int32)],
                                   aehs[h])
            return carry2

        def a_sub(j, carry2):
            cp1 = pltpu.async_copy(asd_hbm.at[sda_v.at[j, 0]], srows_v, sem)
            cp2 = pltpu.async_copy(asd_hbm.at[sda_v.at[j, 1]], drows_v, sem)
            cp1.wait()
            cp2.wait()
            lax.fori_loop(j * (KA // 16), (j + 1) * (KA // 16), a_group, 0)
            pltpu.sync_copy(drows_v, acc_sh.at[sda_v.at[j, 1]], add=True)
            return carry2

        pltpu.sync_copy(sda_hbm.at[pl.ds(wid * NCHUNK + s * 8, 8)], sda_v)
        pltpu.async_copy(aed_hbm.at[pl.ds(ebase * 4, 8 * KA * 4)], aed_v,
                         sem).wait()
        lax.fori_loop(0, 8, a_sub, 0)
        pltpu.sync_copy(ex_v.at[pl.ds(0, 8 * KA * 4)],
                        ex_out.at[pl.ds(ebase * 4, 8 * KA * 4)])
        return carry

    lax.fori_loop(0, NCHUNK // 8, a_super, 0)
    plsc.subcore_barrier()

    pltpu.sync_copy(acc_sh.at[pl.ds(r0, RPT)],
                    nacc_out.at[cid, pl.ds(r0, RPT)])
    pltpu.sync_copy(z128_hbm.at[pl.ds(r0, RPT)], acc_sh.at[pl.ds(r0, RPT)])
    plsc.subcore_barrier()

    def b_super(s, carry):
        ebase = wid * EPT + s * 8 * KB

        def b_group(jg, carry2):
            j = jg // (KB // 16)
            g = jg % (KB // 16)
            for i in range(16):
                eloc = g * 16 + i
                for h in range(HEADS):
                    wb = plsc.load_gather(
                        ex_v, [j * (KB * 4) + g * 64
                               + jnp.full((16,), i * 4 + h, jnp.int32)])
                    for half in range(2):
                        jc = 2 * h + half
                        rv = rows_v[eloc, pl.ds(jc * 16, 16)]
                        rows_v[eloc, pl.ds(jc * 16, 16)] = rv * wb
            return carry2

        def b_sub(j, carry2):
            pltpu.async_copy(xp_hbm.at[sdb_v.at[j, 0]], rows_v, sem).wait()
            lax.fori_loop(j * (KB // 16), (j + 1) * (KB // 16), b_group, 0)
            pltpu.sync_copy(rows_v, acc_sh.at[sdb_v.at[j, 1]], add=True)
            return carry2

        pltpu.sync_copy(sdb_hbm.at[pl.ds(wid * NCHUNK_B + s * 8, 8)], sdb_v)
        pltpu.async_copy(ex_out.at[pl.ds(ebase * 4, 8 * KB * 4)], ex_v,
                         sem).wait()
        lax.fori_loop(0, 8, b_sub, 0)
        return carry

    lax.fori_loop(0, NCHUNK_B // 8, b_super, 0)
    plsc.subcore_barrier()
    pltpu.sync_copy(acc_sh.at[pl.ds(r0, RPT)],
                    part_out.at[cid, pl.ds(r0, RPT)])


def kernel(x, edge_index, edge_attr, batch, W_gat, att_src, att_dst, att_edge,
           W_edge, b_gat, enc_W1, enc_b1, prelu_a, enc_W2, enc_b2, p_W1, p_b1,
           p_W2, p_b2, p_W3, p_b3):
    f32 = jnp.float32
    x_pad = jnp.pad(x, ((0, NPAD - N), (0, 0)))
    src_pad = jnp.pad(edge_index[0], (0, EP - E))
    dummy = N + jnp.arange(EP - E, dtype=jnp.int32) % (NPAD - N)
    dst_pad = jnp.concatenate([edge_index[1], dummy])
    sda = jnp.stack([src_pad.reshape(EP // KA, KA),
                     dst_pad.reshape(EP // KA, KA)], axis=1)
    sdb = jnp.stack([src_pad.reshape(EP // KB, KB),
                     dst_pad.reshape(EP // KB, KB)], axis=1)
    ea_pad = jnp.pad(edge_attr, ((0, EP - E), (0, 0)))
    batch3 = jnp.pad(batch, (0, NPAD - N),
                     constant_values=G).reshape(NBLK, 1, BN)

    rep = jnp.repeat(jnp.eye(HEADS, dtype=f32), C, axis=0)
    aef = rep * att_edge.reshape(-1)[:, None]
    att_mat = jnp.concatenate([rep * att_src.reshape(-1)[:, None],
                               rep * att_dst.reshape(-1)[:, None]], axis=1)
    att_mat128 = jnp.pad(att_mat, ((0, 0), (0, 120)))
    zeros128 = jnp.zeros((NPAD, HID), f32)

    xp, asd16 = pl.pallas_call(
        _pre_body,
        grid=(NBLK,),
        in_specs=[pl.BlockSpec((BN, NODE_DIM), lambda i: (i, 0)),
                  pl.BlockSpec((NODE_DIM, HID), lambda i: (0, 0)),
                  pl.BlockSpec((HID, HID), lambda i: (0, 0))],
        out_specs=[pl.BlockSpec((BN, HID), lambda i: (i, 0)),
                   pl.BlockSpec((BN, HID), lambda i: (i, 0))],
        out_shape=[jax.ShapeDtypeStruct((NPAD, HID), f32),
                   jax.ShapeDtypeStruct((NPAD, HID), f32)],
    )(x_pad, W_gat, att_mat128)

    EB = 2048
    aed = pl.pallas_call(
        _aed_body,
        grid=(EP // EB,),
        in_specs=[pl.BlockSpec((EB, EDGE_DIM), lambda i: (i, 0)),
                  pl.BlockSpec((EDGE_DIM, HID), lambda i: (0, 0)),
                  pl.BlockSpec((HID, HEADS), lambda i: (0, 0))],
        out_specs=pl.BlockSpec((EB, HEADS), lambda i: (i, 0)),
        out_shape=jax.ShapeDtypeStruct((EP, HEADS), f32),
    )(ea_pad, W_edge, aef)

    nacc, part, _ = _sc_edges(sda, sdb, aed.reshape(EP * 4), asd16, xp,
                              zeros128)

    exl, dinv = pl.pallas_call(
        _comb_body,
        grid=(NBLK,),
        in_specs=[pl.BlockSpec((NC, BN, HID), lambda i: (0, i, 0)),
                  pl.BlockSpec((BN, HID), lambda i: (i, 0))],
        out_specs=[pl.BlockSpec((BN, HEADS), lambda i: (i, 0)),
                   pl.BlockSpec((BN, HEADS), lambda i: (i, 0))],
        out_shape=[jax.ShapeDtypeStruct((NPAD, HEADS), f32),
                   jax.ShapeDtypeStruct((NPAD, HEADS), f32)],
    )(nacc, asd16)

    em = jnp.repeat(jnp.eye(HEADS, dtype=f32), C, axis=1)
    z_pad, pred = pl.pallas_call(
        _tail_body,
        grid=(NBLK,),
        in_specs=[pl.BlockSpec((NC, BN, HID), lambda i: (0, i, 0)),
                  pl.BlockSpec((BN, HID), lambda i: (i, 0)),
                  pl.BlockSpec((BN, HEADS), lambda i: (i, 0)),
                  pl.BlockSpec((BN, HEADS), lambda i: (i, 0)),
                  pl.BlockSpec((HEADS, HID), lambda i: (0, 0)),
                  pl.BlockSpec((1, HID), lambda i: (0, 0)),
                  pl.BlockSpec((1, 1, BN), lambda i: (i, 0, 0)),
                  pl.BlockSpec((HID, HID), lambda i: (0, 0)),
                  pl.BlockSpec((1, HID), lambda i: (0, 0)),
                  pl.BlockSpec((1, 1), lambda i: (0, 0)),
                  pl.BlockSpec((HID, LATENT), lambda i: (0, 0)),
                  pl.BlockSpec((1, LATENT), lambda i: (0, 0)),
                  pl.BlockSpec((LATENT, 128), lambda i: (0, 0)),
                  pl.BlockSpec((1, 128), lambda i: (0, 0)),
                  pl.BlockSpec((128, 64), lambda i: (0, 0)),
                  pl.BlockSpec((1, 64), lambda i: (0, 0)),
                  pl.BlockSpec((64, 1), lambda i: (0, 0)),
                  pl.BlockSpec((1, 1), lambda i: (0, 0))],
        out_specs=[pl.BlockSpec((BN, LATENT), lambda i: (i, 0)),
                   pl.BlockSpec((G, 1), lambda i: (0, 0))],
        out_shape=[jax.ShapeDtypeStruct((NPAD, LATENT), f32),
                   jax.ShapeDtypeStruct((G, 1), f32)],
        scratch_shapes=[pltpu.VMEM((G, LATENT), f32),
                        pltpu.VMEM((G, 1), f32)],
    )(part, xp, exl, dinv, em, b_gat.reshape(1, HID), batch3, enc_W1,
      enc_b1.reshape(1, HID), prelu_a.reshape(1, 1), enc_W2,
      enc_b2.reshape(1, LATENT), p_W1, p_b1.reshape(1, 128), p_W2,
      p_b2.reshape(1, 64), p_W3, p_b3.reshape(1, 1))

    return (pred.reshape(G), z_pad[:N])


# speedup vs baseline: 22.1503x; 1.0252x over previous
import functools

import jax
import jax.numpy as jnp
from jax import lax
from jax.experimental import pallas as pl
from jax.experimental.pallas import tpu as pltpu
from jax.experimental.pallas import tpu_sc as plsc

N = 10000
E = 320000
NODE_DIM = 128
EDGE_DIM = 16
HEADS = 4
HID = 128
C = HID // HEADS
LATENT = 32
G = 64

NPAD = 10240
EP = 327680
NC = 2
NS = 16
NW = NC * NS
EPT = EP // NW
KA = 64
NCHUNK = EPT // KA
KB = 128
NCHUNK_B = EPT // KB
BN = 1024
NBLK = NPAD // BN
RPT = NPAD // NS


def _pre_body(x_ref, w_ref, am_ref, xp_ref, asd_ref):
    xp = x_ref[...] @ w_ref[...]
    xp_ref[...] = xp
    asd_ref[...] = xp @ am_ref[...]


def _aed_body(ea_ref, we_ref, aef_ref, aed_ref):
    ve = we_ref[...] @ aef_ref[...]
    aed_ref[...] = ea_ref[...] @ ve


def _comb_body(na_ref, asd_ref, exl_ref, dinv_ref):
    na = na_ref[0] + na_ref[1]
    deg = jnp.maximum(na[:, 12:13], 1.0)
    asd = asd_ref[...]
    al = asd[:, 0:4] + asd[:, 4:8] + na[:, 8:12] / deg
    al = jnp.where(al >= 0, al, 0.2 * al)
    exl = jnp.exp(al)
    exl_ref[...] = exl
    dinv_ref[...] = 1.0 / (na[:, 0:4] + exl + 1e-16)


def _silu(v):
    return v / (1.0 + jnp.exp(-v))


def _tail_body(p_ref, xp_ref, exl_ref, dinv_ref, em_ref, bg_ref, b3_ref,
               w1_ref, b1_ref, pa_ref, w2_ref, b2_ref, pw1_ref, pb1_ref,
               pw2_ref, pb2_ref, pw3_ref, pb3_ref, z_ref, pred_ref, ps_ref,
               cnt_ref):
    i = pl.program_id(0)
    exl = exl_ref[...] @ em_ref[...]
    dinv = dinv_ref[...] @ em_ref[...]
    out = (p_ref[0] + p_ref[1] + xp_ref[...] * exl) * dinv + bg_ref[...]
    h = jnp.where(out > 0, out, jnp.exp(jnp.minimum(out, 0.0)) - 1.0)
    h1 = h @ w1_ref[...] + b1_ref[...]
    h1 = jnp.where(h1 >= 0, h1, pa_ref[...] * h1)
    z = h1 @ w2_ref[...] + b2_ref[...]
    z_ref[...] = z
    bids = b3_ref[0]
    gi = lax.broadcasted_iota(jnp.int32, (G, 1), 0)
    mask = (bids == gi).astype(jnp.float32)
    ps = mask @ z
    ct = jnp.sum(mask, axis=1, keepdims=True)

    @pl.when(i == 0)
    def _():
        ps_ref[...] = ps
        cnt_ref[...] = ct

    @pl.when(i > 0)
    def _():
        ps_ref[...] += ps
        cnt_ref[...] += ct

    @pl.when(i == pl.num_programs(0) - 1)
    def _():
        pooled = ps_ref[...] / jnp.maximum(cnt_ref[...], 1.0)
        p1 = _silu(pooled @ pw1_ref[...] + pb1_ref[...])
        p2 = _silu(p1 @ pw2_ref[...] + pb2_ref[...])
        pred_ref[...] = p2 @ pw3_ref[...] + pb3_ref[...]


_MESH = plsc.VectorSubcoreMesh(core_axis_name="c", subcore_axis_name="s",
                               num_cores=NC, num_subcores=NS)
_SC_PARAMS = pltpu.CompilerParams(needs_layout_passes=False)


@functools.partial(
    pl.kernel,
    out_type=[jax.ShapeDtypeStruct((NC, NPAD, HID), jnp.float32),
              jax.ShapeDtypeStruct((NC, NPAD, HID), jnp.float32),
              jax.ShapeDtypeStruct((EP * 4,), jnp.float32)],
    mesh=_MESH,
    scratch_types=[
        pltpu.VMEM((8, 2, KA), jnp.int32),
        pltpu.VMEM((8, 2, KB), jnp.int32),
        pltpu.VMEM((8 * KA * 4,), jnp.float32),
        pltpu.VMEM((8 * KB * 4,), jnp.float32),
        pltpu.VMEM((KA, HID), jnp.float32),
        pltpu.VMEM((KA, HID), jnp.float32),
        pltpu.VMEM((KB, HID), jnp.float32),
        pltpu.VMEM_SHARED((NPAD, HID), jnp.float32),
        pltpu.SemaphoreType.DMA,
        pltpu.SemaphoreType.DMA,
    ],
    compiler_params=_SC_PARAMS,
)
def _sc_edges(sda_hbm, sdb_hbm, aed_hbm, asd_hbm, xp_hbm, z128_hbm,
              nacc_out, part_out, ex_out,
              sda_v, sdb_v, aed_v, ex_v, srows_v, drows_v, rows_v,
              acc_sh, sem, sem2):
    cid = lax.axis_index("c")
    sid = lax.axis_index("s")
    wid = cid * NS + sid
    r0 = sid * RPT

    i16 = lax.iota(jnp.int32, 16)
    one16 = jnp.ones((16,), jnp.float32)

    pltpu.sync_copy(z128_hbm.at[pl.ds(r0, RPT)], acc_sh.at[pl.ds(r0, RPT)])
    plsc.subcore_barrier()

    def a_super(s, carry):
        ebase = wid * EPT + s * 8 * KA

        def a_group(jg, carry2):
            j = jg // (KA // 16)
            g = jg % (KA // 16)
            l16 = g * 16 + i16
            asrcs = []
            adsts = []
            aehs = []
            for h in range(HEADS):
                asrcs.append(plsc.load_gather(
                    srows_v, [l16, jnp.full((16,), h, jnp.int32)]))
                adsts.append(plsc.load_gather(
                    drows_v, [l16, jnp.full((16,), 4 + h, jnp.int32)]))
                aehs.append(plsc.load_gather(
                    aed_v, [j * (KA * 4) + l16 * 4 + h]))
            plsc.store_scatter(drows_v, [l16, jnp.full((16,), 12, jnp.int32)],
                               one16)
            for h in range(HEADS):
                al = asrcs[h] + adsts[h] + aehs[h]
                al = jnp.where(al >= 0, al, 0.2 * al)
                exh = jnp.exp(al)
                plsc.store_scatter(ex_v, [j * (KA * 4) + l16 * 4 + h], exh)
                plsc.store_scatter(drows_v, [l16, jnp.full((16,), h,
                                                           jnp.int32)], exh)
                plsc.store_scatter(drows_v, [l16, jnp.full((16,), 8 + h,
                                                           jnp.int32)],
                                   aehs[h])
            return carry2

        def a_sub(j, carry2):
            cp1 = pltpu.async_copy(asd_hbm.at[sda_v.at[j, 0]], srows_v, sem)

            @pl.when(j > 0)
            def _():
                pltpu.make_async_copy(z128_hbm.at[pl.ds(0, KA)], drows_v,
                                      sem2).wait()

            cp2 = pltpu.async_copy(asd_hbm.at[sda_v.at[j, 1]], drows_v, sem)
            cp1.wait()
            cp2.wait()
            lax.fori_loop(j * (KA // 16), (j + 1) * (KA // 16), a_group, 0)
            pltpu.async_copy(drows_v, acc_sh.at[sda_v.at[j, 1]], sem2,
                             add=True)
            return carry2

        @pl.when(s > 0)
        def _():
            pltpu.make_async_copy(z128_hbm.at[pl.ds(0, KA)], drows_v,
                                  sem2).wait()

        pltpu.sync_copy(sda_hbm.at[pl.ds(wid * NCHUNK + s * 8, 8)], sda_v)
        pltpu.async_copy(aed_hbm.at[pl.ds(ebase * 4, 8 * KA * 4)], aed_v,
                         sem).wait()
        lax.fori_loop(0, 8, a_sub, 0)
        pltpu.sync_copy(ex_v.at[pl.ds(0, 8 * KA * 4)],
                        ex_out.at[pl.ds(ebase * 4, 8 * KA * 4)])
        return carry

    lax.fori_loop(0, NCHUNK // 8, a_super, 0)
    pltpu.make_async_copy(z128_hbm.at[pl.ds(0, KA)], drows_v, sem2).wait()
    plsc.subcore_barrier()

    pltpu.sync_copy(acc_sh.at[pl.ds(r0, RPT)],
                    nacc_out.at[cid, pl.ds(r0, RPT)])
    pltpu.sync_copy(z128_hbm.at[pl.ds(r0, RPT)], acc_sh.at[pl.ds(r0, RPT)])
    plsc.subcore_barrier()

    def b_super(s, carry):
        ebase = wid * EPT + s * 8 * KB

        def b_group(jg, carry2):
            j = jg // (KB // 16)
            g = jg % (KB // 16)
            for i in range(16):
                eloc = g * 16 + i
                for h in range(HEADS):
                    wb = plsc.load_gather(
                        ex_v, [j * (KB * 4) + g * 64
                               + jnp.full((16,), i * 4 + h, jnp.int32)])
                    for half in range(2):
                        jc = 2 * h + half
                        rv = rows_v[eloc, pl.ds(jc * 16, 16)]
                        rows_v[eloc, pl.ds(jc * 16, 16)] = rv * wb
            return carry2

        def b_sub(j, carry2):
            pltpu.async_copy(xp_hbm.at[sdb_v.at[j, 0]], rows_v, sem).wait()
            lax.fori_loop(j * (KB // 16), (j + 1) * (KB // 16), b_group, 0)
            pltpu.sync_copy(rows_v, acc_sh.at[sdb_v.at[j, 1]], add=True)
            return carry2

        pltpu.sync_copy(sdb_hbm.at[pl.ds(wid * NCHUNK_B + s * 8, 8)], sdb_v)
        pltpu.async_copy(ex_out.at[pl.ds(ebase * 4, 8 * KB * 4)], ex_v,
                         sem).wait()
        lax.fori_loop(0, 8, b_sub, 0)
        return carry

    lax.fori_loop(0, NCHUNK_B // 8, b_super, 0)
    plsc.subcore_barrier()
    pltpu.sync_copy(acc_sh.at[pl.ds(r0, RPT)],
                    part_out.at[cid, pl.ds(r0, RPT)])


def kernel(x, edge_index, edge_attr, batch, W_gat, att_src, att_dst, att_edge,
           W_edge, b_gat, enc_W1, enc_b1, prelu_a, enc_W2, enc_b2, p_W1, p_b1,
           p_W2, p_b2, p_W3, p_b3):
    f32 = jnp.float32
    x_pad = jnp.pad(x, ((0, NPAD - N), (0, 0)))
    src_pad = jnp.pad(edge_index[0], (0, EP - E))
    dummy = N + jnp.arange(EP - E, dtype=jnp.int32) % (NPAD - N)
    dst_pad = jnp.concatenate([edge_index[1], dummy])
    sda = jnp.stack([src_pad.reshape(EP // KA, KA),
                     dst_pad.reshape(EP // KA, KA)], axis=1)
    sdb = jnp.stack([src_pad.reshape(EP // KB, KB),
                     dst_pad.reshape(EP // KB, KB)], axis=1)
    ea_pad = jnp.pad(edge_attr, ((0, EP - E), (0, 0)))
    batch3 = jnp.pad(batch, (0, NPAD - N),
                     constant_values=G).reshape(NBLK, 1, BN)

    rep = jnp.repeat(jnp.eye(HEADS, dtype=f32), C, axis=0)
    aef = rep * att_edge.reshape(-1)[:, None]
    att_mat = jnp.concatenate([rep * att_src.reshape(-1)[:, None],
                               rep * att_dst.reshape(-1)[:, None]], axis=1)
    att_mat128 = jnp.pad(att_mat, ((0, 0), (0, 120)))
    zeros128 = jnp.zeros((NPAD, HID), f32)

    xp, asd16 = pl.pallas_call(
        _pre_body,
        grid=(NBLK,),
        in_specs=[pl.BlockSpec((BN, NODE_DIM), lambda i: (i, 0)),
                  pl.BlockSpec((NODE_DIM, HID), lambda i: (0, 0)),
                  pl.BlockSpec((HID, HID), lambda i: (0, 0))],
        out_specs=[pl.BlockSpec((BN, HID), lambda i: (i, 0)),
                   pl.BlockSpec((BN, HID), lambda i: (i, 0))],
        out_shape=[jax.ShapeDtypeStruct((NPAD, HID), f32),
                   jax.ShapeDtypeStruct((NPAD, HID), f32)],
    )(x_pad, W_gat, att_mat128)

    EB = 2048
    aed = pl.pallas_call(
        _aed_body,
        grid=(EP // EB,),
        in_specs=[pl.BlockSpec((EB, EDGE_DIM), lambda i: (i, 0)),
                  pl.BlockSpec((EDGE_DIM, HID), lambda i: (0, 0)),
                  pl.BlockSpec((HID, HEADS), lambda i: (0, 0))],
        out_specs=pl.BlockSpec((EB, HEADS), lambda i: (i, 0)),
        out_shape=jax.ShapeDtypeStruct((EP, HEADS), f32),
    )(ea_pad, W_edge, aef)

    nacc, part, _ = _sc_edges(sda, sdb, aed.reshape(EP * 4), asd16, xp,
                              zeros128)

    exl, dinv = pl.pallas_call(
        _comb_body,
        grid=(NBLK,),
        in_specs=[pl.BlockSpec((NC, BN, HID), lambda i: (0, i, 0)),
                  pl.BlockSpec((BN, HID), lambda i: (i, 0))],
        out_specs=[pl.BlockSpec((BN, HEADS), lambda i: (i, 0)),
                   pl.BlockSpec((BN, HEADS), lambda i: (i, 0))],
        out_shape=[jax.ShapeDtypeStruct((NPAD, HEADS), f32),
                   jax.ShapeDtypeStruct((NPAD, HEADS), f32)],
    )(nacc, asd16)

    em = jnp.repeat(jnp.eye(HEADS, dtype=f32), C, axis=1)
    z_pad, pred = pl.pallas_call(
        _tail_body,
        grid=(NBLK,),
        in_specs=[pl.BlockSpec((NC, BN, HID), lambda i: (0, i, 0)),
                  pl.BlockSpec((BN, HID), lambda i: (i, 0)),
                  pl.BlockSpec((BN, HEADS), lambda i: (i, 0)),
                  pl.BlockSpec((BN, HEADS), lambda i: (i, 0)),
                  pl.BlockSpec((HEADS, HID), lambda i: (0, 0)),
                  pl.BlockSpec((1, HID), lambda i: (0, 0)),
                  pl.BlockSpec((1, 1, BN), lambda i: (i, 0, 0)),
                  pl.BlockSpec((HID, HID), lambda i: (0, 0)),
                  pl.BlockSpec((1, HID), lambda i: (0, 0)),
                  pl.BlockSpec((1, 1), lambda i: (0, 0)),
                  pl.BlockSpec((HID, LATENT), lambda i: (0, 0)),
                  pl.BlockSpec((1, LATENT), lambda i: (0, 0)),
                  pl.BlockSpec((LATENT, 128), lambda i: (0, 0)),
                  pl.BlockSpec((1, 128), lambda i: (0, 0)),
                  pl.BlockSpec((128, 64), lambda i: (0, 0)),
                  pl.BlockSpec((1, 64), lambda i: (0, 0)),
                  pl.BlockSpec((64, 1), lambda i: (0, 0)),
                  pl.BlockSpec((1, 1), lambda i: (0, 0))],
        out_specs=[pl.BlockSpec((BN, LATENT), lambda i: (i, 0)),
                   pl.BlockSpec((G, 1), lambda i: (0, 0))],
        out_shape=[jax.ShapeDtypeStruct((NPAD, LATENT), f32),
                   jax.ShapeDtypeStruct((G, 1), f32)],
        scratch_shapes=[pltpu.VMEM((G, LATENT), f32),
                        pltpu.VMEM((G, 1), f32)],
    )(part, xp, exl, dinv, em, b_gat.reshape(1, HID), batch3, enc_W1,
      enc_b1.reshape(1, HID), prelu_a.reshape(1, 1), enc_W2,
      enc_b2.reshape(1, LATENT), p_W1, p_b1.reshape(1, 128), p_W2,
      p_b2.reshape(1, 64), p_W3, p_b3.reshape(1, 1))

    return (pred.reshape(G), z_pad[:N])
